# Initial kernel scaffold; baseline (speedup 1.0000x reference)
import jax
import jax.numpy as jnp
from jax.experimental import pallas as pl


def kernel(x, weight):
    raise NotImplementedError("write your pallas kernel here")


# speedup vs baseline: 4.7571x; 4.7571x over previous
import jax
import jax.numpy as jnp
from jax.experimental import pallas as pl

_BLK = 256


def _bcast_copy_kernel(w_ref, o_ref):
    o_ref[...] = jnp.broadcast_to(w_ref[...][None], o_ref.shape)


def kernel(x, weight):
    batch, seq = x.shape
    dim = weight.shape[1]
    return pl.pallas_call(
        _bcast_copy_kernel,
        grid=(seq // _BLK,),
        in_specs=[pl.BlockSpec((_BLK, dim), lambda j: (j, 0))],
        out_specs=pl.BlockSpec((batch, _BLK, dim), lambda j: (0, j, 0)),
        out_shape=jax.ShapeDtypeStruct((batch, seq, dim), weight.dtype),
    )(weight)


# speedup vs baseline: 5.0397x; 1.0594x over previous
import jax
import jax.numpy as jnp
from jax.experimental import pallas as pl
from jax.experimental.pallas import tpu as pltpu

_BLK = 512


def _bcast_copy_kernel(w_ref, o_ref):
    o_ref[...] = jnp.broadcast_to(w_ref[...][None], o_ref.shape)


def kernel(x, weight):
    batch, seq = x.shape
    dim = weight.shape[1]
    return pl.pallas_call(
        _bcast_copy_kernel,
        grid=(seq // _BLK,),
        in_specs=[pl.BlockSpec((_BLK, dim), lambda j: (j, 0))],
        out_specs=pl.BlockSpec((batch, _BLK, dim), lambda j: (0, j, 0)),
        out_shape=jax.ShapeDtypeStruct((batch, seq, dim), weight.dtype),
        compiler_params=pltpu.CompilerParams(
            dimension_semantics=("parallel",),
        ),
    )(weight)


# speedup vs baseline: 5.1875x; 1.0293x over previous
import jax
import jax.numpy as jnp
from jax.experimental import pallas as pl
from jax.experimental.pallas import tpu as pltpu

_BLK = 1024


def _bcast_copy_kernel(w_ref, o_ref):
    o_ref[...] = jnp.broadcast_to(w_ref[...][None], o_ref.shape)


def kernel(x, weight):
    batch, seq = x.shape
    dim = weight.shape[1]
    return pl.pallas_call(
        _bcast_copy_kernel,
        grid=(seq // _BLK,),
        in_specs=[pl.BlockSpec((_BLK, dim), lambda j: (j, 0))],
        out_specs=pl.BlockSpec((batch, _BLK, dim), lambda j: (0, j, 0)),
        out_shape=jax.ShapeDtypeStruct((batch, seq, dim), weight.dtype),
        compiler_params=pltpu.CompilerParams(
            dimension_semantics=("parallel",),
        ),
    )(weight)
